# baseline (speedup 1.0000x reference)
import functools

import jax
import jax.numpy as jnp
from jax import lax
from jax.experimental import pallas as pl
from jax.experimental.pallas import tpu as pltpu
from jax.experimental.pallas import tpu_sc as plsc

N = 10000
E = 320000
POS = 3
FEATS = 128
EDIM = 4
MDIM = 16
XPD = FEATS + 16
OUTW = 2 * MDIM
HID = 2 * (2 * FEATS + EDIM + 1)

_NC = 2
_NS = 16
_NW = _NC * _NS
_EW = E // _NW
_GK = 80
_GCH = _EW // _GK
_RPT = N // _NS

def _gather_body(table, row2, col2, gr, gc, idxr, idxc,
                 br0, br1, bc0, bc1,
                 gsr0, gsr1, gsc0, gsc1, ssr0, ssr1, ssc0, ssc1):
    c = lax.axis_index("c")
    s = lax.axis_index("s")
    w = s * _NC + c
    cbase = w * _GCH
    ebase = w * _EW
    pltpu.sync_copy(row2.at[pl.ds(cbase, _GCH)], idxr)
    pltpu.sync_copy(col2.at[pl.ds(cbase, _GCH)], idxc)
    brs = (br0, br1)
    bcs = (bc0, bc1)
    gss = ((gsr0, gsc0), (gsr1, gsc1))
    sss = ((ssr0, ssc0), (ssr1, ssc1))

    def fire(i, b):
        pltpu.async_copy(table.at[idxr.at[i]], brs[b], gss[b][0])
        pltpu.async_copy(table.at[idxc.at[i]], bcs[b], gss[b][1])

    def wait_gather(b):
        pltpu.make_async_copy(table.at[pl.ds(0, _GK)], brs[b], gss[b][0]).wait()
        pltpu.make_async_copy(table.at[pl.ds(0, _GK)], bcs[b], gss[b][1]).wait()

    def fire_store(i, b):
        off = ebase + i * _GK
        pltpu.async_copy(brs[b], gr.at[pl.ds(off, _GK)], sss[b][0])
        pltpu.async_copy(bcs[b], gc.at[pl.ds(off, _GK)], sss[b][1])

    def wait_store(b):
        pltpu.make_async_copy(brs[b], gr.at[pl.ds(0, _GK)], sss[b][0]).wait()
        pltpu.make_async_copy(bcs[b], gc.at[pl.ds(0, _GK)], sss[b][1]).wait()

    fire(0, 0)

    def pair(p, carry):
        i0 = 2 * p

        @pl.when(p > 0)
        def _():
            wait_store(1)

        fire(i0 + 1, 1)
        wait_gather(0)
        fire_store(i0, 0)
        wait_store(0)
        fire(i0 + 2, 0)
        wait_gather(1)
        fire_store(i0 + 1, 1)
        return carry

    lax.fori_loop(0, (_GCH - 1) // 2, pair, 0)
    wait_gather(0)
    fire_store(_GCH - 1, 0)
    wait_store(1)
    wait_store(0)


@functools.cache
def _gather_kernel():
    return pl.kernel(
        _gather_body,
        out_type=[
            jax.ShapeDtypeStruct((E, XPD), jnp.float32),
            jax.ShapeDtypeStruct((E, XPD), jnp.float32),
        ],
        mesh=plsc.VectorSubcoreMesh(core_axis_name="c", subcore_axis_name="s"),
        compiler_params=pltpu.CompilerParams(use_tc_tiling_on_sc=False),
        scratch_types=[
            pltpu.VMEM((_GCH, _GK), jnp.int32),
            pltpu.VMEM((_GCH, _GK), jnp.int32),
            pltpu.VMEM((_GK, XPD), jnp.float32),
            pltpu.VMEM((_GK, XPD), jnp.float32),
            pltpu.VMEM((_GK, XPD), jnp.float32),
            pltpu.VMEM((_GK, XPD), jnp.float32),
            pltpu.SemaphoreType.DMA,
            pltpu.SemaphoreType.DMA,
            pltpu.SemaphoreType.DMA,
            pltpu.SemaphoreType.DMA,
            pltpu.SemaphoreType.DMA,
            pltpu.SemaphoreType.DMA,
            pltpu.SemaphoreType.DMA,
            pltpu.SemaphoreType.DMA,
        ],
    )


def _gather(xp, row2, col2):
    return _gather_kernel()(xp, row2, col2)


def _scatter_body(vals, row2, zz, out, idx2, v0, v1, ls0, ls1, acc):
    c = lax.axis_index("c")
    s = lax.axis_index("s")
    w = s * _NC + c
    cbase = w * _GCH
    ebase = w * _EW
    pltpu.sync_copy(zz.at[pl.ds(s * _RPT, _RPT)], acc.at[pl.ds(s * _RPT, _RPT)])
    pltpu.sync_copy(row2.at[pl.ds(cbase, _GCH)], idx2)
    plsc.subcore_barrier()
    vbufs = (v0, v1)
    lsems = (ls0, ls1)

    def fire_load(i, b):
        pltpu.async_copy(vals.at[pl.ds(ebase + i * _GK, _GK)], vbufs[b], lsems[b])

    def wait_load(b):
        pltpu.make_async_copy(vals.at[pl.ds(0, _GK)], vbufs[b], lsems[b]).wait()

    def scat(i, b):
        pltpu.sync_copy(vbufs[b], acc.at[idx2.at[i]], add=True)

    fire_load(0, 0)

    def pair(p, carry):
        i0 = 2 * p
        fire_load(i0 + 1, 1)
        wait_load(0)
        scat(i0, 0)
        fire_load(i0 + 2, 0)
        wait_load(1)
        scat(i0 + 1, 1)
        return carry

    lax.fori_loop(0, (_GCH - 1) // 2, pair, 0)
    wait_load(0)
    scat(_GCH - 1, 0)
    plsc.subcore_barrier()
    pltpu.sync_copy(acc.at[pl.ds(s * _RPT, _RPT)],
                    out.at[c, pl.ds(s * _RPT, _RPT)])


@functools.cache
def _scatter_kernel():
    return pl.kernel(
        _scatter_body,
        out_type=jax.ShapeDtypeStruct((_NC, N, OUTW), jnp.float32),
        mesh=plsc.VectorSubcoreMesh(core_axis_name="c", subcore_axis_name="s"),
        compiler_params=pltpu.CompilerParams(use_tc_tiling_on_sc=False),
        scratch_types=[
            pltpu.VMEM((_GCH, _GK), jnp.int32),
            pltpu.VMEM((_GK, OUTW), jnp.float32),
            pltpu.VMEM((_GK, OUTW), jnp.float32),
            pltpu.SemaphoreType.DMA,
            pltpu.SemaphoreType.DMA,
            pltpu.VMEM_SHARED((N, OUTW), jnp.float32),
        ],
    )


def _scatter(eout, row2, zz):
    return _scatter_kernel()(eout, row2, zz)


_BE = 2000


def _silu(x):
    return x / (1.0 + jnp.exp(-x))


def _edge_body(gr, gc, ea, wa, wb, wc, be1, we2, be2, wc1, bc1, wc2, bc2, out):
    fr = gr[:, :FEATS]
    fc = gc[:, :FEATS]
    rel = gr[:, FEATS:] - gc[:, FEATS:]
    rd = jnp.sum(rel * rel, axis=1, keepdims=True)
    lane = lax.broadcasted_iota(jnp.int32, (_BE, 16), 1)
    eax = ea[...] + jnp.where(lane == EDIM, rd, 0.0)
    pre = (jnp.dot(fr.astype(jnp.bfloat16), wa[...].astype(jnp.bfloat16),
                   preferred_element_type=jnp.float32)
           + jnp.dot(fc.astype(jnp.bfloat16), wb[...].astype(jnp.bfloat16),
                     preferred_element_type=jnp.float32)
           + jnp.dot(eax, wc[...], preferred_element_type=jnp.float32)
           + be1[...])
    h = _silu(pre)
    m = _silu(jnp.dot(h, we2[...], preferred_element_type=jnp.float32) + be2[...])
    cwh = _silu(jnp.dot(m, wc1[...], preferred_element_type=jnp.float32) + bc1[...])
    cw = jnp.dot(cwh, wc2[...], preferred_element_type=jnp.float32) + bc2[...]
    out[...] = jnp.concatenate([m, rel * cw], axis=1)


def _edge_call(gr, gc, ea16, wa, wb, wc, be1, we2, be2, wc1, bc1, wc2, bc2):
    full = lambda r, c: pl.BlockSpec((r, c), lambda i: (0, 0))
    return pl.pallas_call(
        _edge_body,
        grid=(E // _BE,),
        in_specs=[
            pl.BlockSpec((_BE, XPD), lambda i: (i, 0)),
            pl.BlockSpec((_BE, XPD), lambda i: (i, 0)),
            pl.BlockSpec((_BE, 16), lambda i: (i, 0)),
            full(FEATS, HID), full(FEATS, HID), full(16, HID), full(1, HID),
            full(HID, MDIM), full(1, MDIM),
            full(MDIM, 4 * MDIM), full(1, 4 * MDIM),
            full(4 * MDIM, 16), full(1, 16),
        ],
        out_specs=pl.BlockSpec((_BE, OUTW), lambda i: (i, 0)),
        out_shape=jax.ShapeDtypeStruct((E, OUTW), jnp.float32),
    )(gr, gc, ea16, wa, wb, wc, be1, we2, be2, wc1, bc1, wc2, bc2)


_BN = 2000


def _node_body(xp, a0, a1, lng, lnb, wn1a, wn1b, bn1, wn2, bn2, out):
    f = xp[:, :FEATS]
    c16 = xp[:, FEATS:]
    agg = a0[...] + a1[...]
    m_i = agg[:, :MDIM]
    cd = agg[:, MDIM:]
    mu = jnp.mean(f, axis=1, keepdims=True)
    var = jnp.mean((f - mu) ** 2, axis=1, keepdims=True)
    fl = (f - mu) * lax.rsqrt(var + 1e-5) * lng[...] + lnb[...]
    nh = _silu(jnp.dot(fl, wn1a[...], preferred_element_type=jnp.float32)
               + jnp.dot(m_i, wn1b[...], preferred_element_type=jnp.float32)
               + bn1[...])
    nh = jnp.dot(nh, wn2[...], preferred_element_type=jnp.float32) + bn2[...]
    out[...] = jnp.concatenate([f + nh, c16 + cd], axis=1)


def _node_call(xp, a0, a1, lng, lnb, wn1a, wn1b, bn1, wn2, bn2):
    full = lambda r, c: pl.BlockSpec((r, c), lambda i: (0, 0))
    return pl.pallas_call(
        _node_body,
        grid=(N // _BN,),
        in_specs=[
            pl.BlockSpec((_BN, XPD), lambda i: (i, 0)),
            pl.BlockSpec((_BN, OUTW), lambda i: (i, 0)),
            pl.BlockSpec((_BN, OUTW), lambda i: (i, 0)),
            full(1, FEATS), full(1, FEATS),
            full(FEATS, 2 * FEATS), full(MDIM, 2 * FEATS), full(1, 2 * FEATS),
            full(2 * FEATS, FEATS), full(1, FEATS),
        ],
        out_specs=pl.BlockSpec((_BN, XPD), lambda i: (i, 0)),
        out_shape=jax.ShapeDtypeStruct((N, XPD), jnp.float32),
    )(xp, a0, a1, lng, lnb, wn1a, wn1b, bn1, wn2, bn2)


def kernel(x, edge_index, batch, edge_attr, params):
    del batch
    row2 = edge_index[0].astype(jnp.int32).reshape(E // _GK, _GK)
    col2 = edge_index[1].astype(jnp.int32).reshape(E // _GK, _GK)
    ea16 = jnp.pad(edge_attr.astype(jnp.float32), ((0, 0), (0, 16 - EDIM)))
    xp = jnp.concatenate(
        [x[:, POS:], x[:, :POS], jnp.zeros((N, XPD - FEATS - POS), jnp.float32)],
        axis=1)
    zz = jnp.zeros((N, OUTW), jnp.float32)

    for p in params:
        we1 = p['We1']
        wa = we1[:FEATS]
        wb = we1[FEATS:2 * FEATS]
        wc = (jnp.zeros((16, HID), jnp.float32)
              .at[:EDIM].set(we1[2 * FEATS:2 * FEATS + EDIM])
              .at[EDIM].set(we1[-1]))
        gr, gc = _gather(xp, row2, col2)
        eout = _edge_call(
            gr, gc, ea16, wa, wb, wc, p['be1'][None],
            p['We2'], p['be2'][None], p['Wc1'], p['bc1'][None],
            jnp.tile(p['Wc2'], (1, 16)), jnp.tile(p['bc2'][None], (1, 16)))
        partials = _scatter(eout, row2, zz)
        xp = _node_call(
            xp, partials[0], partials[1], p['ln_g'][None], p['ln_b'][None],
            p['Wn1'][:FEATS], p['Wn1'][FEATS:], p['bn1'][None],
            p['Wn2'], p['bn2'][None])

    return jnp.concatenate([xp[:, FEATS:FEATS + POS], xp[:, :FEATS]], axis=1)


# speedup vs baseline: 1.0356x; 1.0356x over previous
import functools

import jax
import jax.numpy as jnp
from jax import lax
from jax.experimental import pallas as pl
from jax.experimental.pallas import tpu as pltpu
from jax.experimental.pallas import tpu_sc as plsc

N = 10000
E = 320000
POS = 3
FEATS = 128
EDIM = 4
MDIM = 16
XPD = FEATS + 16
OUTW = 2 * MDIM
HID = 2 * (2 * FEATS + EDIM + 1)

_NC = 2
_NS = 16
_NW = _NC * _NS
_EW = E // _NW
_GK = 80
_GCH = _EW // _GK
_RPT = N // _NS
_S = 5
_ES = E // _S
_GCHS = _GCH // _S
_EWS = _GCHS * _GK
_CROWS = _ES // _GK

def _gather_body(table, row2, col2, gr, gc, idxr, idxc,
                 br0, br1, bc0, bc1,
                 gsr0, gsr1, gsc0, gsc1, ssr0, ssr1, ssc0, ssc1):
    c = lax.axis_index("c")
    s = lax.axis_index("s")
    w = s * _NC + c
    cbase = w * _GCHS
    ebase = w * _EWS
    pltpu.sync_copy(row2.at[pl.ds(cbase, _GCHS)], idxr)
    pltpu.sync_copy(col2.at[pl.ds(cbase, _GCHS)], idxc)
    brs = (br0, br1)
    bcs = (bc0, bc1)
    gss = ((gsr0, gsc0), (gsr1, gsc1))
    sss = ((ssr0, ssc0), (ssr1, ssc1))

    def fire(i, b):
        pltpu.async_copy(table.at[idxr.at[i]], brs[b], gss[b][0])
        pltpu.async_copy(table.at[idxc.at[i]], bcs[b], gss[b][1])

    def wait_gather(b):
        pltpu.make_async_copy(table.at[pl.ds(0, _GK)], brs[b], gss[b][0]).wait()
        pltpu.make_async_copy(table.at[pl.ds(0, _GK)], bcs[b], gss[b][1]).wait()

    def fire_store(i, b):
        off = ebase + i * _GK
        pltpu.async_copy(brs[b], gr.at[pl.ds(off, _GK)], sss[b][0])
        pltpu.async_copy(bcs[b], gc.at[pl.ds(off, _GK)], sss[b][1])

    def wait_store(b):
        pltpu.make_async_copy(brs[b], gr.at[pl.ds(0, _GK)], sss[b][0]).wait()
        pltpu.make_async_copy(bcs[b], gc.at[pl.ds(0, _GK)], sss[b][1]).wait()

    fire(0, 0)

    def pair(p, carry):
        i0 = 2 * p

        @pl.when(p > 0)
        def _():
            wait_store(1)

        fire(i0 + 1, 1)
        wait_gather(0)
        fire_store(i0, 0)
        wait_store(0)
        fire(i0 + 2, 0)
        wait_gather(1)
        fire_store(i0 + 1, 1)
        return carry

    lax.fori_loop(0, (_GCHS - 1) // 2, pair, 0)
    wait_gather(0)
    fire_store(_GCHS - 1, 0)
    wait_store(1)
    wait_store(0)


@functools.cache
def _gather_kernel():
    return pl.kernel(
        _gather_body,
        out_type=[
            jax.ShapeDtypeStruct((_ES, XPD), jnp.float32),
            jax.ShapeDtypeStruct((_ES, XPD), jnp.float32),
        ],
        mesh=plsc.VectorSubcoreMesh(core_axis_name="c", subcore_axis_name="s"),
        compiler_params=pltpu.CompilerParams(use_tc_tiling_on_sc=False),
        scratch_types=[
            pltpu.VMEM((_GCHS, _GK), jnp.int32),
            pltpu.VMEM((_GCHS, _GK), jnp.int32),
            pltpu.VMEM((_GK, XPD), jnp.float32),
            pltpu.VMEM((_GK, XPD), jnp.float32),
            pltpu.VMEM((_GK, XPD), jnp.float32),
            pltpu.VMEM((_GK, XPD), jnp.float32),
            pltpu.SemaphoreType.DMA,
            pltpu.SemaphoreType.DMA,
            pltpu.SemaphoreType.DMA,
            pltpu.SemaphoreType.DMA,
            pltpu.SemaphoreType.DMA,
            pltpu.SemaphoreType.DMA,
            pltpu.SemaphoreType.DMA,
            pltpu.SemaphoreType.DMA,
        ],
    )


def _gather(xp, row2_slice, col2_slice):
    return _gather_kernel()(xp, row2_slice, col2_slice)


def _scatter_body(va, vb, vc, vd, ve, row2, zz, out, idx2, v0, v1, ls0, ls1, acc):
    c = lax.axis_index("c")
    s = lax.axis_index("s")
    w = s * _NC + c
    pltpu.sync_copy(zz.at[pl.ds(s * _RPT, _RPT)], acc.at[pl.ds(s * _RPT, _RPT)])
    plsc.subcore_barrier()
    vbufs = (v0, v1)
    lsems = (ls0, ls1)
    ebase = w * _EWS

    for q, vals in enumerate((va, vb, vc, vd, ve)):
        pltpu.sync_copy(row2.at[pl.ds(q * _CROWS + w * _GCHS, _GCHS)], idx2)

        def fire_load(i, b, vals=vals):
            pltpu.async_copy(vals.at[pl.ds(ebase + i * _GK, _GK)],
                             vbufs[b], lsems[b])

        def wait_load(b, vals=vals):
            pltpu.make_async_copy(vals.at[pl.ds(0, _GK)],
                                  vbufs[b], lsems[b]).wait()

        def scat(i, b):
            pltpu.sync_copy(vbufs[b], acc.at[idx2.at[i]], add=True)

        fire_load(0, 0)

        def pair(p, carry):
            i0 = 2 * p
            fire_load(i0 + 1, 1)
            wait_load(0)
            scat(i0, 0)
            fire_load(i0 + 2, 0)
            wait_load(1)
            scat(i0 + 1, 1)
            return carry

        lax.fori_loop(0, (_GCHS - 1) // 2, pair, 0)
        wait_load(0)
        scat(_GCHS - 1, 0)

    plsc.subcore_barrier()
    pltpu.sync_copy(acc.at[pl.ds(s * _RPT, _RPT)],
                    out.at[c, pl.ds(s * _RPT, _RPT)])


@functools.cache
def _scatter_kernel():
    return pl.kernel(
        _scatter_body,
        out_type=jax.ShapeDtypeStruct((_NC, N, OUTW), jnp.float32),
        mesh=plsc.VectorSubcoreMesh(core_axis_name="c", subcore_axis_name="s"),
        compiler_params=pltpu.CompilerParams(use_tc_tiling_on_sc=False),
        scratch_types=[
            pltpu.VMEM((_GCHS, _GK), jnp.int32),
            pltpu.VMEM((_GK, OUTW), jnp.float32),
            pltpu.VMEM((_GK, OUTW), jnp.float32),
            pltpu.SemaphoreType.DMA,
            pltpu.SemaphoreType.DMA,
            pltpu.VMEM_SHARED((N, OUTW), jnp.float32),
        ],
    )


def _scatter(eouts, row2, zz):
    return _scatter_kernel()(*eouts, row2, zz)


_BE = 2000


def _silu(x):
    return x / (1.0 + jnp.exp(-x))


def _edge_body(gr, gc, ea, wa, wb, wc, be1, we2, be2, wc1, bc1, wc2, bc2, out):
    fr = gr[:, :FEATS]
    fc = gc[:, :FEATS]
    rel = gr[:, FEATS:] - gc[:, FEATS:]
    rd = jnp.sum(rel * rel, axis=1, keepdims=True)
    lane = lax.broadcasted_iota(jnp.int32, (_BE, 16), 1)
    eax = ea[...] + jnp.where(lane == EDIM, rd, 0.0)
    pre = (jnp.dot(fr, wa[...], preferred_element_type=jnp.float32)
           + jnp.dot(fc, wb[...], preferred_element_type=jnp.float32)
           + jnp.dot(eax, wc[...], preferred_element_type=jnp.float32)
           + be1[...])
    h = _silu(pre)
    m = _silu(jnp.dot(h, we2[...], preferred_element_type=jnp.float32) + be2[...])
    cwh = _silu(jnp.dot(m, wc1[...], preferred_element_type=jnp.float32) + bc1[...])
    cw = jnp.dot(cwh, wc2[...], preferred_element_type=jnp.float32) + bc2[...]
    out[...] = jnp.concatenate([m, rel * cw], axis=1)


def _edge_call(gr, gc, ea16, wa, wb, wc, be1, we2, be2, wc1, bc1, wc2, bc2):
    full = lambda r, c: pl.BlockSpec((r, c), lambda i: (0, 0))
    ne = gr.shape[0]
    return pl.pallas_call(
        _edge_body,
        grid=(ne // _BE,),
        in_specs=[
            pl.BlockSpec((_BE, XPD), lambda i: (i, 0)),
            pl.BlockSpec((_BE, XPD), lambda i: (i, 0)),
            pl.BlockSpec((_BE, 16), lambda i: (i, 0)),
            full(FEATS, HID), full(FEATS, HID), full(16, HID), full(1, HID),
            full(HID, MDIM), full(1, MDIM),
            full(MDIM, 4 * MDIM), full(1, 4 * MDIM),
            full(4 * MDIM, 16), full(1, 16),
        ],
        out_specs=pl.BlockSpec((_BE, OUTW), lambda i: (i, 0)),
        out_shape=jax.ShapeDtypeStruct((ne, OUTW), jnp.float32),
    )(gr, gc, ea16, wa, wb, wc, be1, we2, be2, wc1, bc1, wc2, bc2)


_BN = 2000


def _node_body(xp, a0, a1, lng, lnb, wn1a, wn1b, bn1, wn2, bn2, out):
    f = xp[:, :FEATS]
    c16 = xp[:, FEATS:]
    agg = a0[...] + a1[...]
    m_i = agg[:, :MDIM]
    cd = agg[:, MDIM:]
    mu = jnp.mean(f, axis=1, keepdims=True)
    var = jnp.mean((f - mu) ** 2, axis=1, keepdims=True)
    fl = (f - mu) * lax.rsqrt(var + 1e-5) * lng[...] + lnb[...]
    nh = _silu(jnp.dot(fl, wn1a[...], preferred_element_type=jnp.float32)
               + jnp.dot(m_i, wn1b[...], preferred_element_type=jnp.float32)
               + bn1[...])
    nh = jnp.dot(nh, wn2[...], preferred_element_type=jnp.float32) + bn2[...]
    out[...] = jnp.concatenate([f + nh, c16 + cd], axis=1)


def _node_call(xp, a0, a1, lng, lnb, wn1a, wn1b, bn1, wn2, bn2):
    full = lambda r, c: pl.BlockSpec((r, c), lambda i: (0, 0))
    return pl.pallas_call(
        _node_body,
        grid=(N // _BN,),
        in_specs=[
            pl.BlockSpec((_BN, XPD), lambda i: (i, 0)),
            pl.BlockSpec((_BN, OUTW), lambda i: (i, 0)),
            pl.BlockSpec((_BN, OUTW), lambda i: (i, 0)),
            full(1, FEATS), full(1, FEATS),
            full(FEATS, 2 * FEATS), full(MDIM, 2 * FEATS), full(1, 2 * FEATS),
            full(2 * FEATS, FEATS), full(1, FEATS),
        ],
        out_specs=pl.BlockSpec((_BN, XPD), lambda i: (i, 0)),
        out_shape=jax.ShapeDtypeStruct((N, XPD), jnp.float32),
    )(xp, a0, a1, lng, lnb, wn1a, wn1b, bn1, wn2, bn2)


def kernel(x, edge_index, batch, edge_attr, params):
    del batch
    row2 = edge_index[0].astype(jnp.int32).reshape(E // _GK, _GK)
    col2 = edge_index[1].astype(jnp.int32).reshape(E // _GK, _GK)
    ea16 = jnp.pad(edge_attr.astype(jnp.float32), ((0, 0), (0, 16 - EDIM)))
    xp = jnp.concatenate(
        [x[:, POS:], x[:, :POS], jnp.zeros((N, XPD - FEATS - POS), jnp.float32)],
        axis=1)
    zz = jnp.zeros((N, OUTW), jnp.float32)

    for p in params:
        we1 = p['We1']
        wa = we1[:FEATS]
        wb = we1[FEATS:2 * FEATS]
        wc = (jnp.zeros((16, HID), jnp.float32)
              .at[:EDIM].set(we1[2 * FEATS:2 * FEATS + EDIM])
              .at[EDIM].set(we1[-1]))
        gathered = {0: _gather(xp, row2[:_CROWS], col2[:_CROWS])}
        eouts = []
        for q in range(_S):
            if q + 1 < _S:
                gathered[q + 1] = _gather(
                    xp,
                    row2[(q + 1) * _CROWS:(q + 2) * _CROWS],
                    col2[(q + 1) * _CROWS:(q + 2) * _CROWS])
            gr, gc = gathered.pop(q)
            eouts.append(_edge_call(
                gr, gc, ea16[q * _ES:(q + 1) * _ES], wa, wb, wc, p['be1'][None],
                p['We2'], p['be2'][None], p['Wc1'], p['bc1'][None],
                jnp.tile(p['Wc2'], (1, 16)), jnp.tile(p['bc2'][None], (1, 16))))
        partials = _scatter(eouts, row2, zz)
        xp = _node_call(
            xp, partials[0], partials[1], p['ln_g'][None], p['ln_b'][None],
            p['Wn1'][:FEATS], p['Wn1'][FEATS:], p['bn1'][None],
            p['Wn2'], p['bn2'][None])

    return jnp.concatenate([xp[:, FEATS:FEATS + POS], xp[:, :FEATS]], axis=1)
